# baseline (speedup 1.0000x reference)
import jax
import jax.numpy as jnp
from jax.experimental import pallas as pl
from jax.experimental.pallas import tpu as pltpu

N = 65536
DIM = 256
CPAD = 128
BLK = 4096
NBLK = N // BLK


def _body(feat_ref, gt_ref, out_ref, acc_ref, cnt_ref):
    i = pl.program_id(0)

    @pl.when(i == 0)
    def _init():
        acc_ref[...] = jnp.zeros_like(acc_ref)
        cnt_ref[...] = jnp.zeros_like(cnt_ref)

    x = feat_ref[...]
    inv_norm = jax.lax.rsqrt(jnp.sum(x * x, axis=1))

    gt = gt_ref[0]
    class_ids = jax.lax.broadcasted_iota(jnp.int32, (CPAD, BLK), 0)
    eq = class_ids == gt
    oh_scaled = jnp.where(eq, inv_norm[None, :], 0.0)

    acc_ref[...] += jnp.dot(oh_scaled, x, preferred_element_type=jnp.float32)
    cnt_ref[...] += jnp.broadcast_to(
        jnp.sum(eq.astype(jnp.float32), axis=1, keepdims=True), cnt_ref.shape
    )

    @pl.when(i == NBLK - 1)
    def _finalize():
        counts = cnt_ref[:, 0:1]
        means = acc_ref[...] / jnp.maximum(counts, 1.0)
        nrm = jnp.sqrt(jnp.sum(means * means, axis=1, keepdims=True))
        mem_bank = means / jnp.maximum(nrm, 1e-12)
        out_ref[...] = (0.0 * jnp.sum(mem_bank)).reshape(1, 1)


def kernel(fpn_feat, cat_gt, cat_score_pred, cnt_score_pred, prototypes, branch):
    gt3 = cat_gt.astype(jnp.int32).reshape(NBLK, 1, BLK)
    out = pl.pallas_call(
        _body,
        grid=(NBLK,),
        in_specs=[
            pl.BlockSpec((BLK, DIM), lambda i: (i, 0)),
            pl.BlockSpec((1, 1, BLK), lambda i: (i, 0, 0)),
        ],
        out_specs=pl.BlockSpec((1, 1), lambda i: (0, 0)),
        out_shape=jax.ShapeDtypeStruct((1, 1), jnp.float32),
        scratch_shapes=[
            pltpu.VMEM((CPAD, DIM), jnp.float32),
            pltpu.VMEM((CPAD, 128), jnp.float32),
        ],
        compiler_params=pltpu.CompilerParams(
            dimension_semantics=("arbitrary",),
        ),
    )(fpn_feat, gt3)
    return out[0, 0]


# speedup vs baseline: 1.1581x; 1.1581x over previous
import jax
import jax.numpy as jnp
from jax.experimental import pallas as pl
from jax.experimental.pallas import tpu as pltpu

N = 65536
DIM = 256
CPAD = 128
BLK = 8192
NBLK = N // BLK


def _body(feat_ref, gt_ref, out_ref, acc_ref, cnt_ref):
    i = pl.program_id(0)

    @pl.when(i == 0)
    def _init():
        acc_ref[...] = jnp.zeros_like(acc_ref)
        cnt_ref[...] = jnp.zeros_like(cnt_ref)

    x = feat_ref[...]
    inv_norm = jax.lax.rsqrt(jnp.sum(x * x, axis=1))

    gt = gt_ref[0]
    class_ids = jax.lax.broadcasted_iota(jnp.int32, (CPAD, BLK), 0)
    eq = class_ids == gt
    oh_scaled = jnp.where(eq, inv_norm[None, :], 0.0)

    acc_ref[...] += jnp.dot(oh_scaled, x, preferred_element_type=jnp.float32)
    cnt_ref[...] += jnp.broadcast_to(
        jnp.sum(eq.astype(jnp.float32), axis=1, keepdims=True), cnt_ref.shape
    )

    @pl.when(i == NBLK - 1)
    def _finalize():
        counts = cnt_ref[:, 0:1]
        means = acc_ref[...] / jnp.maximum(counts, 1.0)
        nrm = jnp.sqrt(jnp.sum(means * means, axis=1, keepdims=True))
        mem_bank = means / jnp.maximum(nrm, 1e-12)
        out_ref[...] = (0.0 * jnp.sum(mem_bank)).reshape(1, 1)


def kernel(fpn_feat, cat_gt, cat_score_pred, cnt_score_pred, prototypes, branch):
    gt3 = cat_gt.astype(jnp.int32).reshape(NBLK, 1, BLK)
    out = pl.pallas_call(
        _body,
        grid=(NBLK,),
        in_specs=[
            pl.BlockSpec((BLK, DIM), lambda i: (i, 0)),
            pl.BlockSpec((1, 1, BLK), lambda i: (i, 0, 0)),
        ],
        out_specs=pl.BlockSpec((1, 1), lambda i: (0, 0)),
        out_shape=jax.ShapeDtypeStruct((1, 1), jnp.float32),
        scratch_shapes=[
            pltpu.VMEM((CPAD, DIM), jnp.float32),
            pltpu.VMEM((CPAD, 128), jnp.float32),
        ],
        compiler_params=pltpu.CompilerParams(
            dimension_semantics=("arbitrary",),
        ),
    )(fpn_feat, gt3)
    return out[0, 0]
